# baseline (speedup 1.0000x reference)
import functools

import jax
import jax.numpy as jnp
from jax import lax
from jax.experimental import pallas as pl
from jax.experimental.pallas import tpu as pltpu
from jax.experimental.pallas import tpu_sc as plsc

N = 10000
E = 160000
TW = 5
NV = 2
NS = 1
EMB = 164
L = 6

DP = 256
SP = 128
NP = N + 8
NC = 2
NSUB = 16
NW = NC * NSUB
CH = 128
CHG = 64
EPAD = 163840
EPW = EPAD // NW
NCHUNK = EPW // CH
NCHUNKG = EPW // CHG
BE = 4096
BN = 2000
NB = N // BN


def _silu(v):
    return v * jax.nn.sigmoid(v)


def _dot(a, b):
    return jnp.dot(a, b, preferred_element_type=jnp.float32)


@functools.lru_cache(maxsize=None)
def _sc_kernels(epad):
    mesh = plsc.VectorSubcoreMesh(core_axis_name="c", subcore_axis_name="s")
    epw = epad // NW
    nchg = epw // CHG
    nch = epw // CH

    @functools.partial(
        pl.kernel,
        mesh=mesh,
        compiler_params=pltpu.CompilerParams(needs_layout_passes=False),
        out_type=jax.ShapeDtypeStruct((epad, SP), jnp.uint32),
        scratch_types=[
            pltpu.VMEM((nchg, CHG), jnp.int32),
            pltpu.VMEM((nchg, CHG), jnp.int32),
            pltpu.VMEM((4, CHG, SP), jnp.uint32),
            pltpu.VMEM((4, CHG, SP), jnp.uint32),
            pltpu.SemaphoreType.DMA,
            pltpu.SemaphoreType.DMA,
            pltpu.SemaphoreType.DMA,
            pltpu.SemaphoreType.DMA,
            pltpu.SemaphoreType.DMA,
            pltpu.SemaphoreType.DMA,
            pltpu.SemaphoreType.DMA,
            pltpu.SemaphoreType.DMA,
        ],
    )
    def sc_gather(a_hbm, b_hbm, dst2_hbm, src2_hbm, g_hbm,
                  dsti, srci, bufa, bufb,
                  sa0, sb0, sa1, sb1, sa2, sb2, sa3, sb3):
        c = lax.axis_index("c")
        s = lax.axis_index("s")
        wid = s * NC + c
        base = wid * epw
        sems_a = (sa0, sa1, sa2, sa3)
        sems_b = (sb0, sb1, sb2, sb3)
        pltpu.sync_copy(dst2_hbm.at[wid], dsti)
        pltpu.sync_copy(src2_hbm.at[wid], srci)

        def fire(k, p):
            pltpu.async_copy(a_hbm.at[dsti.at[k]], bufa.at[p], sems_a[p])
            pltpu.async_copy(b_hbm.at[srci.at[k]], bufb.at[p], sems_b[p])

        def process(k, p):
            pltpu.make_async_copy(
                a_hbm.at[dsti.at[k]], bufa.at[p], sems_a[p]).wait()
            pltpu.make_async_copy(
                b_hbm.at[srci.at[k]], bufb.at[p], sems_b[p]).wait()

            def add_row(r2, carry):
                for rr in range(2):
                    r = 2 * r2 + rr
                    for j in range(SP // 16):
                        sl = pl.ds(j * 16, 16)
                        va = plsc.bitcast(bufa[p, r, sl], jnp.bfloat16)
                        vb = plsc.bitcast(bufb[p, r, sl], jnp.bfloat16)
                        bufa[p, r, sl] = plsc.bitcast(va + vb, jnp.uint32)
                return carry

            lax.fori_loop(0, CHG // 2, add_row, 0)
            pltpu.sync_copy(bufa.at[p],
                            g_hbm.at[pl.ds(base + k * CHG, CHG)])

        fire(0, 0)
        fire(1, 1)
        fire(2, 2)
        last = nchg // 4 - 1

        def body(k4, carry):
            e = 4 * k4
            fire(e + 3, 3)
            process(e, 0)

            @pl.when(k4 < last)
            def _f0():
                fire(e + 4, 0)

            process(e + 1, 1)

            @pl.when(k4 < last)
            def _f1():
                fire(e + 5, 1)

            process(e + 2, 2)

            @pl.when(k4 < last)
            def _f2():
                fire(e + 6, 2)

            process(e + 3, 3)
            return carry

        lax.fori_loop(0, nchg // 4, body, 0)

    @functools.partial(
        pl.kernel,
        mesh=mesh,
        out_type=jax.ShapeDtypeStruct((NC, 2, NP, SP), jnp.float32),
        scratch_types=[
            pltpu.VMEM((nch, CH), jnp.int32),
            pltpu.VMEM((2, CH, SP), jnp.float32),
            pltpu.VMEM_SHARED((NP, SP), jnp.float32),
            pltpu.SemaphoreType.DMA,
            pltpu.SemaphoreType.DMA,
        ],
    )
    def sc_scatter(s1_hbm, s2_hbm, dst2s_hbm, zeros_hbm, out_hbm,
                   idxv, bufs, shared, se0, se1):
        c = lax.axis_index("c")
        s = lax.axis_index("s")
        wid = s * NC + c
        base = wid * epw
        sems = (se0, se1)
        pltpu.sync_copy(dst2s_hbm.at[wid], idxv)
        for src_hbm, pp in ((s1_hbm, 0), (s2_hbm, 1)):
            @pl.when(s == 0)
            def _zero():
                pltpu.sync_copy(zeros_hbm, shared)

            plsc.subcore_barrier()

            def fire(k, q, src_hbm=src_hbm):
                pltpu.async_copy(src_hbm.at[pl.ds(base + k * CH, CH)],
                                 bufs.at[q], sems[q])

            def process(k, q, src_hbm=src_hbm):
                pltpu.make_async_copy(
                    src_hbm.at[pl.ds(base + k * CH, CH)],
                    bufs.at[q], sems[q]).wait()
                pltpu.sync_copy(bufs.at[q], shared.at[idxv.at[k]], add=True)

            fire(0, 0)

            def body(k2, carry):
                e = 2 * k2
                fire(e + 1, 1)
                process(e, 0)

                @pl.when(k2 < nch // 2 - 1)
                def _refire():
                    fire(e + 2, 0)

                process(e + 1, 1)
                return carry

            lax.fori_loop(0, nch // 2, body, 0)
            plsc.subcore_barrier()

            @pl.when(s == 0)
            def _dump(pp=pp):
                pltpu.sync_copy(shared, out_hbm.at[c, pp])

    return sc_gather, sc_scatter


def _max_body(pos_ref, out_ref):
    out_ref[...] = jnp.max(pos_ref[...]).reshape(1, 1)


def _pack(v):
    vb = v.astype(jnp.bfloat16)
    lo = jax.lax.bitcast_convert_type(vb[:, :SP], jnp.uint16
                                      ).astype(jnp.uint32)
    hi = jax.lax.bitcast_convert_type(vb[:, SP:], jnp.uint16
                                      ).astype(jnp.uint32)
    return lo | (hi << jnp.uint32(16))


def _pre_body(x_ref, pos_ref, time_ref, vars_ref, pmax_ref, w1_ref, b1_ref,
              w2_ref, b2_ref, whd_ref, wxa_ref, bm1_ref, whs_ref, wxb_ref,
              ni_ref, h_ref, a_ref, b_ref):
    pos_n = pos_ref[...] / pmax_ref[...]
    ni = jnp.concatenate(
        [x_ref[...], pos_n, time_ref[...] * (1.0 / 10.0), vars_ref[...]],
        axis=1)
    h = _silu(_dot(ni, w1_ref[...]) + b1_ref[...])
    h = _silu(_dot(h, w2_ref[...]) + b2_ref[...])
    ni_ref[...] = ni
    h_ref[...] = h
    a_ref[...] = _pack(_dot(h, whd_ref[...]) + _dot(ni, wxa_ref[...])
                       + bm1_ref[...])
    b_ref[...] = _pack(_dot(h, whs_ref[...]) + _dot(ni, wxb_ref[...]))


def _unpk(bits16):
    return jax.lax.bitcast_convert_type(
        bits16.astype(jnp.uint16), jnp.bfloat16).astype(jnp.float32)


def _msg_body(g_ref, w2a_ref, w2b_ref, b2_ref, s1_ref, s2_ref):
    u = g_ref[...]
    tl = _silu(_unpk(u & jnp.uint32(0xFFFF)))
    th = _silu(_unpk(u >> jnp.uint32(16)))
    sv = _silu(_dot(tl, w2a_ref[...]) + _dot(th, w2b_ref[...])
               + b2_ref[...])
    col = lax.broadcasted_iota(jnp.int32, sv.shape, 1)
    sv = jnp.where(col == EMB, 1.0, sv)
    s1_ref[...] = sv[:, :SP]
    s2_ref[...] = sv[:, SP:2 * SP]


NSEG = 2


def _upd_body(*refs):
    h_ref = refs[0]
    nagg = 2 * NSEG
    los = refs[1:1 + nagg]
    his = refs[1 + nagg:1 + 2 * nagg]
    (ni_ref, u1h_ref, u1a_ref, wxu_ref, ub1_ref, u2_ref, ub2_ref,
     hr_ref, st_ref) = refs[1 + 2 * nagg:]
    j = pl.program_id(0)
    h = h_ref[...]
    dcol = EMB - SP
    lo = los[0][...]
    for r in los[1:]:
        lo = lo + r[...]
    hi = his[0][...]
    for r in his[1:]:
        hi = hi + r[...]
    deg = jnp.clip(hi[:, dcol:dcol + 1], 1.0, None)
    agg = jnp.concatenate([lo, hi[:, :dcol]], axis=1) / deg
    u = _silu(_dot(h, u1h_ref[...]) + _dot(agg, u1a_ref[...]) +
              _dot(ni_ref[...], wxu_ref[...]) + ub1_ref[...])
    u = _silu(_dot(u, u2_ref[...]) + ub2_ref[...])
    hr = h + u
    hr_ref[...] = hr

    @pl.when(j == 0)
    def _init():
        st_ref[...] = jnp.zeros_like(st_ref)

    st_ref[...] += jnp.concatenate(
        [jnp.sum(hr, axis=0, keepdims=True),
         jnp.sum(hr * hr, axis=0, keepdims=True)], axis=0)


def _norm(hr_ref, st_ref):
    mean = st_ref[0:1, :] * (1.0 / N)
    ex2 = st_ref[1:2, :] * (1.0 / N)
    varr = ex2 - mean * mean
    return (hr_ref[...] - mean) * lax.rsqrt(varr + 1e-5)


def _ab_body(hr_ref, st_ref, ni_ref, whd_ref, wxa_ref, bm1_ref, whs_ref,
             wxb_ref, hn_ref, an_ref, bn_ref):
    hn = _norm(hr_ref, st_ref)
    ni = ni_ref[...]
    hn_ref[...] = hn
    an_ref[...] = _pack(_dot(hn, whd_ref[...]) + _dot(ni, wxa_ref[...])
                        + bm1_ref[...])
    bn_ref[...] = _pack(_dot(hn, whs_ref[...]) + _dot(ni, wxb_ref[...]))


def _dec_body(hr_ref, st_ref, ni_ref, w1m_ref, b1m_ref, w2m_ref, db2_ref,
              dt_ref, out_ref):
    hn = _norm(hr_ref, st_ref)
    c1f = _silu(_dot(hn, w1m_ref[...]) + b1m_ref[...])
    diff = _dot(c1f, w2m_ref[...]) + db2_ref[...]
    steps = (lax.broadcasted_iota(jnp.int32, (1, TW), 1) + 1
             ).astype(jnp.float32)
    dtv = dt_ref[...] * steps
    out_ref[...] = ni_ref[:, TW - 1:TW] + dtv * diff


def _pre_call(x, pos, time, variables, *weights):
    pmax = pl.pallas_call(
        _max_body,
        out_shape=jax.ShapeDtypeStruct((1, 1), jnp.float32),
    )(pos)
    nrow = lambda w: pl.BlockSpec((BN, w), lambda j: (j, 0))
    full = lambda a, b: pl.BlockSpec((a, b), lambda j: (0, 0))
    return pl.pallas_call(
        _pre_body,
        grid=(NB,),
        in_specs=[
            nrow(TW), nrow(1), nrow(1), nrow(NV - 1), full(1, 1),
            full(TW + NS + NV, EMB), full(1, EMB), full(EMB, EMB),
            full(1, EMB), full(EMB, DP), full(TW + NS + NV, DP), full(1, DP),
            full(EMB, DP), full(TW + NS + NV, DP),
        ],
        out_specs=(nrow(TW + NS + NV), nrow(EMB), nrow(SP), nrow(SP)),
        out_shape=(jax.ShapeDtypeStruct((N, TW + NS + NV), jnp.float32),
                   jax.ShapeDtypeStruct((N, EMB), jnp.float32),
                   jax.ShapeDtypeStruct((N, SP), jnp.uint32),
                   jax.ShapeDtypeStruct((N, SP), jnp.uint32)),
    )(x, pos, time, variables, pmax, *weights)


def _msg_call(g, w2a, w2b, b2):
    nblk = g.shape[0] // BE
    return pl.pallas_call(
        _msg_body,
        grid=(nblk,),
        in_specs=[
            pl.BlockSpec((BE, SP), lambda i: (i, 0)),
            pl.BlockSpec((SP, DP), lambda i: (0, 0)),
            pl.BlockSpec((SP, DP), lambda i: (0, 0)),
            pl.BlockSpec((1, DP), lambda i: (0, 0)),
        ],
        out_specs=(pl.BlockSpec((BE, SP), lambda i: (i, 0)),
                   pl.BlockSpec((BE, SP), lambda i: (i, 0))),
        out_shape=(jax.ShapeDtypeStruct((g.shape[0], SP), jnp.float32),
                   jax.ShapeDtypeStruct((g.shape[0], SP), jnp.float32)),
    )(g, w2a, w2b, b2)


def _upd_call(h, los, his, ni, u1h, u1a, wxu, ub1, u2, ub2):
    nrow = lambda w: pl.BlockSpec((BN, w), lambda j: (j, 0))
    full = lambda a, b: pl.BlockSpec((a, b), lambda j: (0, 0))
    nagg = 2 * NSEG
    return pl.pallas_call(
        _upd_body,
        grid=(NB,),
        in_specs=[nrow(EMB)] + [nrow(SP)] * (2 * nagg) + [
            nrow(TW + NS + NV),
            full(EMB, EMB), full(EMB, EMB), full(TW + NS + NV, EMB),
            full(1, EMB), full(EMB, EMB), full(1, EMB),
        ],
        out_specs=(nrow(EMB), full(2, EMB)),
        out_shape=(jax.ShapeDtypeStruct((N, EMB), jnp.float32),
                   jax.ShapeDtypeStruct((2, EMB), jnp.float32)),
    )(h, *los, *his, ni, u1h, u1a, wxu, ub1, u2, ub2)


def _ab_call(hr, st, ni, whd, wxa, bm1, whs, wxb):
    nrow = lambda w: pl.BlockSpec((BN, w), lambda j: (j, 0))
    full = lambda a, b: pl.BlockSpec((a, b), lambda j: (0, 0))
    return pl.pallas_call(
        _ab_body,
        grid=(NB,),
        in_specs=[
            nrow(EMB), full(2, EMB), nrow(TW + NS + NV),
            full(EMB, DP), full(TW + NS + NV, DP), full(1, DP),
            full(EMB, DP), full(TW + NS + NV, DP),
        ],
        out_specs=(nrow(EMB), nrow(SP), nrow(SP)),
        out_shape=(jax.ShapeDtypeStruct((N, EMB), jnp.float32),
                   jax.ShapeDtypeStruct((N, SP), jnp.uint32),
                   jax.ShapeDtypeStruct((N, SP), jnp.uint32)),
    )(hr, st, ni, whd, wxa, bm1, whs, wxb)


def _dec_call(hr, st, ni, w1m, b1m, w2m, db2, dt2):
    nrow = lambda w: pl.BlockSpec((BN, w), lambda j: (j, 0))
    full = lambda a, b: pl.BlockSpec((a, b), lambda j: (0, 0))
    return pl.pallas_call(
        _dec_body,
        grid=(NB,),
        in_specs=[
            nrow(EMB), full(2, EMB), nrow(TW + NS + NV),
            full(EMB, 400), full(1, 400), full(400, TW),
            full(1, 1), full(1, 1),
        ],
        out_specs=nrow(TW),
        out_shape=jax.ShapeDtypeStruct((N, TW), jnp.float32),
    )(hr, st, ni, w1m, b1m, w2m, db2, dt2)


def kernel(x, pos, time, variables, batch, edge_index, dt, enc_W1, enc_b1,
           enc_W2, enc_b2, msg1_W, msg1_b, msg2_W, msg2_b, upd1_W, upd1_b,
           upd2_W, upd2_b, dec1_W, dec1_b, dec2_W, dec2_b):
    f32 = jnp.float32
    padc = lambda w: jnp.pad(w, [(0, 0)] * (w.ndim - 1) + [(0, DP - EMB)])
    Whd = padc(msg1_W[:, :EMB, :])
    Whs = padc(msg1_W[:, EMB:2 * EMB, :])
    WxA = padc(msg1_W[:, 2 * EMB:, :])
    WxB = padc(jnp.concatenate(
        [-msg1_W[:, 2 * EMB:2 * EMB + TW + NS, :],
         jnp.zeros((L, NV, EMB), f32)], axis=1))
    bm1 = padc(msg1_b)[:, None, :]
    W2p = jnp.pad(msg2_W, ((0, 0), (0, DP - EMB), (0, DP - EMB)))
    b2p = padc(msg2_b)[:, None, :]
    U1h = upd1_W[:, :EMB, :]
    U1a = upd1_W[:, EMB:2 * EMB, :]
    WxU = jnp.concatenate(
        [jnp.zeros((L, TW + NS, EMB), f32), upd1_W[:, 2 * EMB:, :]], axis=1)
    ub1 = upd1_b[:, None, :]
    ub2 = upd2_b[:, None, :]
    o_id = jnp.repeat(jnp.arange(8), 50)
    j_id = jnp.tile(jnp.arange(50), 8)
    fgrid = jnp.arange(EMB)[:, None]
    k1 = fgrid - 3 * j_id[None, :]
    W1m = jnp.where((k1 >= 0) & (k1 < 16),
                    dec1_W[o_id[None, :], 0, jnp.clip(k1, 0, 15)], 0.0)
    b1m = dec1_b[o_id][None, :]
    k2 = j_id[:, None] - 9 * jnp.arange(5)[None, :]
    W2m = jnp.where((k2 >= 0) & (k2 < 14),
                    dec2_W[0, o_id[:, None], jnp.clip(k2, 0, 13)], 0.0)
    db2 = dec2_b[None, :]
    dt2 = dt[None, :]

    src = edge_index[0]
    dst = edge_index[1]
    EPH = EPAD // NSEG
    src_g = jnp.pad(src, (0, EPAD - E)).reshape(EPAD // CHG, CHG)
    dst_g = jnp.pad(dst, (0, EPAD - E)).reshape(EPAD // CHG, CHG)
    dst_s = jnp.pad(dst, (0, EPAD - E),
                    constant_values=N).reshape(EPAD // CH, CH)
    hg = EPH // CHG
    hs = EPH // CH
    r3 = lambda a, n: a.reshape(NW, n // NW, a.shape[-1])
    src_gs = [r3(src_g[q * hg:(q + 1) * hg], hg) for q in range(NSEG)]
    dst_gs = [r3(dst_g[q * hg:(q + 1) * hg], hg) for q in range(NSEG)]
    dst_ss = [r3(dst_s[q * hs:(q + 1) * hs], hs) for q in range(NSEG)]
    zeros = jnp.zeros((NP, SP), f32)

    ni, h, A, B = _pre_call(x, pos, time, variables, enc_W1, enc_b1[None, :],
                            enc_W2, enc_b2[None, :], Whd[0], WxA[0], bm1[0],
                            Whs[0], WxB[0])
    sc_gather, sc_scatter = _sc_kernels(EPH)
    out = None
    for i in range(L):
        gs = [sc_gather(A, B, dst_gs[q], src_gs[q]) for q in range(NSEG)]
        los, his = [], []
        for q in range(NSEG):
            s1q, s2q = _msg_call(gs[q], W2p[i, :SP], W2p[i, SP:], b2p[i])
            agq = sc_scatter(s1q, s2q, dst_ss[q], zeros)
            los += [agq[0, 0], agq[1, 0]]
            his += [agq[0, 1], agq[1, 1]]
        hr, st = _upd_call(h, los, his,
                           ni, U1h[i], U1a[i], WxU[i], ub1[i], upd2_W[i],
                           ub2[i])
        if i < L - 1:
            h, A, B = _ab_call(hr, st, ni, Whd[i + 1], WxA[i + 1], bm1[i + 1],
                               Whs[i + 1], WxB[i + 1])
        else:
            out = _dec_call(hr, st, ni, W1m, b1m, W2m, db2, dt2)
    return out


# speedup vs baseline: 1.0115x; 1.0115x over previous
import functools

import jax
import jax.numpy as jnp
from jax import lax
from jax.experimental import pallas as pl
from jax.experimental.pallas import tpu as pltpu
from jax.experimental.pallas import tpu_sc as plsc

N = 10000
E = 160000
TW = 5
NV = 2
NS = 1
EMB = 164
L = 6

DP = 256
SP = 128
NP = N + 8
NC = 2
NSUB = 16
NW = NC * NSUB
CH = 128
CHG = 64
EPAD = 163840
EPW = EPAD // NW
NCHUNK = EPW // CH
NCHUNKG = EPW // CHG
BE = 2048
BN = 2000
NB = N // BN


def _silu(v):
    return v * jax.nn.sigmoid(v)


def _dot(a, b):
    return jnp.dot(a, b, preferred_element_type=jnp.float32)


@functools.lru_cache(maxsize=None)
def _sc_kernels(epad):
    mesh = plsc.VectorSubcoreMesh(core_axis_name="c", subcore_axis_name="s")
    epw = epad // NW
    nchg = epw // CHG
    nch = epw // CH

    @functools.partial(
        pl.kernel,
        mesh=mesh,
        compiler_params=pltpu.CompilerParams(needs_layout_passes=False),
        out_type=jax.ShapeDtypeStruct((epad, SP), jnp.uint32),
        scratch_types=[
            pltpu.VMEM((nchg, CHG), jnp.int32),
            pltpu.VMEM((nchg, CHG), jnp.int32),
            pltpu.VMEM((4, CHG, SP), jnp.uint32),
            pltpu.VMEM((4, CHG, SP), jnp.uint32),
            pltpu.SemaphoreType.DMA,
            pltpu.SemaphoreType.DMA,
            pltpu.SemaphoreType.DMA,
            pltpu.SemaphoreType.DMA,
            pltpu.SemaphoreType.DMA,
            pltpu.SemaphoreType.DMA,
            pltpu.SemaphoreType.DMA,
            pltpu.SemaphoreType.DMA,
        ],
    )
    def sc_gather(a_hbm, b_hbm, dst2_hbm, src2_hbm, g_hbm,
                  dsti, srci, bufa, bufb,
                  sa0, sb0, sa1, sb1, sa2, sb2, sa3, sb3):
        c = lax.axis_index("c")
        s = lax.axis_index("s")
        wid = s * NC + c
        base = wid * epw
        sems_a = (sa0, sa1, sa2, sa3)
        sems_b = (sb0, sb1, sb2, sb3)
        pltpu.sync_copy(dst2_hbm.at[wid], dsti)
        pltpu.sync_copy(src2_hbm.at[wid], srci)

        def fire(k, p):
            pltpu.async_copy(a_hbm.at[dsti.at[k]], bufa.at[p], sems_a[p])
            pltpu.async_copy(b_hbm.at[srci.at[k]], bufb.at[p], sems_b[p])

        def process(k, p):
            pltpu.make_async_copy(
                a_hbm.at[dsti.at[k]], bufa.at[p], sems_a[p]).wait()
            pltpu.make_async_copy(
                b_hbm.at[srci.at[k]], bufb.at[p], sems_b[p]).wait()

            def add_row(r2, carry):
                for rr in range(2):
                    r = 2 * r2 + rr
                    for j in range(SP // 16):
                        sl = pl.ds(j * 16, 16)
                        va = plsc.bitcast(bufa[p, r, sl], jnp.bfloat16)
                        vb = plsc.bitcast(bufb[p, r, sl], jnp.bfloat16)
                        bufa[p, r, sl] = plsc.bitcast(va + vb, jnp.uint32)
                return carry

            lax.fori_loop(0, CHG // 2, add_row, 0)
            pltpu.sync_copy(bufa.at[p],
                            g_hbm.at[pl.ds(base + k * CHG, CHG)])

        fire(0, 0)
        fire(1, 1)
        fire(2, 2)
        last = nchg // 4 - 1

        def body(k4, carry):
            e = 4 * k4
            fire(e + 3, 3)
            process(e, 0)

            @pl.when(k4 < last)
            def _f0():
                fire(e + 4, 0)

            process(e + 1, 1)

            @pl.when(k4 < last)
            def _f1():
                fire(e + 5, 1)

            process(e + 2, 2)

            @pl.when(k4 < last)
            def _f2():
                fire(e + 6, 2)

            process(e + 3, 3)
            return carry

        lax.fori_loop(0, nchg // 4, body, 0)

    @functools.partial(
        pl.kernel,
        mesh=mesh,
        out_type=jax.ShapeDtypeStruct((NC, 2, NP, SP), jnp.float32),
        scratch_types=[
            pltpu.VMEM((nch, CH), jnp.int32),
            pltpu.VMEM((2, CH, SP), jnp.float32),
            pltpu.VMEM_SHARED((NP, SP), jnp.float32),
            pltpu.SemaphoreType.DMA,
            pltpu.SemaphoreType.DMA,
        ],
    )
    def sc_scatter(s1_hbm, s2_hbm, dst2s_hbm, zeros_hbm, out_hbm,
                   idxv, bufs, shared, se0, se1):
        c = lax.axis_index("c")
        s = lax.axis_index("s")
        wid = s * NC + c
        base = wid * epw
        sems = (se0, se1)
        pltpu.sync_copy(dst2s_hbm.at[wid], idxv)
        for src_hbm, pp in ((s1_hbm, 0), (s2_hbm, 1)):
            @pl.when(s == 0)
            def _zero():
                pltpu.sync_copy(zeros_hbm, shared)

            plsc.subcore_barrier()

            def fire(k, q, src_hbm=src_hbm):
                pltpu.async_copy(src_hbm.at[pl.ds(base + k * CH, CH)],
                                 bufs.at[q], sems[q])

            def process(k, q, src_hbm=src_hbm):
                pltpu.make_async_copy(
                    src_hbm.at[pl.ds(base + k * CH, CH)],
                    bufs.at[q], sems[q]).wait()
                pltpu.sync_copy(bufs.at[q], shared.at[idxv.at[k]], add=True)

            fire(0, 0)

            def body(k2, carry):
                e = 2 * k2
                fire(e + 1, 1)
                process(e, 0)

                @pl.when(k2 < nch // 2 - 1)
                def _refire():
                    fire(e + 2, 0)

                process(e + 1, 1)
                return carry

            lax.fori_loop(0, nch // 2, body, 0)
            plsc.subcore_barrier()

            @pl.when(s == 0)
            def _dump(pp=pp):
                pltpu.sync_copy(shared, out_hbm.at[c, pp])

    return sc_gather, sc_scatter


def _max_body(pos_ref, out_ref):
    out_ref[...] = jnp.max(pos_ref[...]).reshape(1, 1)


def _pack(v):
    vb = v.astype(jnp.bfloat16)
    lo = jax.lax.bitcast_convert_type(vb[:, :SP], jnp.uint16
                                      ).astype(jnp.uint32)
    hi = jax.lax.bitcast_convert_type(vb[:, SP:], jnp.uint16
                                      ).astype(jnp.uint32)
    return lo | (hi << jnp.uint32(16))


def _pre_body(x_ref, pos_ref, time_ref, vars_ref, pmax_ref, w1_ref, b1_ref,
              w2_ref, b2_ref, whd_ref, wxa_ref, bm1_ref, whs_ref, wxb_ref,
              ni_ref, h_ref, a_ref, b_ref):
    pos_n = pos_ref[...] / pmax_ref[...]
    ni = jnp.concatenate(
        [x_ref[...], pos_n, time_ref[...] * (1.0 / 10.0), vars_ref[...]],
        axis=1)
    h = _silu(_dot(ni, w1_ref[...]) + b1_ref[...])
    h = _silu(_dot(h, w2_ref[...]) + b2_ref[...])
    ni_ref[...] = ni
    h_ref[...] = h
    a_ref[...] = _pack(_dot(h, whd_ref[...]) + _dot(ni, wxa_ref[...])
                       + bm1_ref[...])
    b_ref[...] = _pack(_dot(h, whs_ref[...]) + _dot(ni, wxb_ref[...]))


def _unpk(bits16):
    return jax.lax.bitcast_convert_type(
        bits16.astype(jnp.uint16), jnp.bfloat16).astype(jnp.float32)


def _msg_body(g_ref, w2a_ref, w2b_ref, b2_ref, s1_ref, s2_ref):
    u = g_ref[...]
    tl = _silu(_unpk(u & jnp.uint32(0xFFFF)))
    th = _silu(_unpk(u >> jnp.uint32(16)))
    sv = _silu(_dot(tl, w2a_ref[...]) + _dot(th, w2b_ref[...])
               + b2_ref[...])
    col = lax.broadcasted_iota(jnp.int32, sv.shape, 1)
    sv = jnp.where(col == EMB, 1.0, sv)
    s1_ref[...] = sv[:, :SP]
    s2_ref[...] = sv[:, SP:2 * SP]


NSEG = 2


def _upd_body(*refs):
    h_ref = refs[0]
    nagg = 2 * NSEG
    los = refs[1:1 + nagg]
    his = refs[1 + nagg:1 + 2 * nagg]
    (ni_ref, u1h_ref, u1a_ref, wxu_ref, ub1_ref, u2_ref, ub2_ref,
     hr_ref, st_ref) = refs[1 + 2 * nagg:]
    j = pl.program_id(0)
    h = h_ref[...]
    dcol = EMB - SP
    lo = los[0][...]
    for r in los[1:]:
        lo = lo + r[...]
    hi = his[0][...]
    for r in his[1:]:
        hi = hi + r[...]
    deg = jnp.clip(hi[:, dcol:dcol + 1], 1.0, None)
    agg = jnp.concatenate([lo, hi[:, :dcol]], axis=1) / deg
    u = _silu(_dot(h, u1h_ref[...]) + _dot(agg, u1a_ref[...]) +
              _dot(ni_ref[...], wxu_ref[...]) + ub1_ref[...])
    u = _silu(_dot(u, u2_ref[...]) + ub2_ref[...])
    hr = h + u
    hr_ref[...] = hr

    @pl.when(j == 0)
    def _init():
        st_ref[...] = jnp.zeros_like(st_ref)

    st_ref[...] += jnp.concatenate(
        [jnp.sum(hr, axis=0, keepdims=True),
         jnp.sum(hr * hr, axis=0, keepdims=True)], axis=0)


def _norm(hr_ref, st_ref):
    mean = st_ref[0:1, :] * (1.0 / N)
    ex2 = st_ref[1:2, :] * (1.0 / N)
    varr = ex2 - mean * mean
    return (hr_ref[...] - mean) * lax.rsqrt(varr + 1e-5)


def _ab_body(hr_ref, st_ref, ni_ref, whd_ref, wxa_ref, bm1_ref, whs_ref,
             wxb_ref, hn_ref, an_ref, bn_ref):
    hn = _norm(hr_ref, st_ref)
    ni = ni_ref[...]
    hn_ref[...] = hn
    an_ref[...] = _pack(_dot(hn, whd_ref[...]) + _dot(ni, wxa_ref[...])
                        + bm1_ref[...])
    bn_ref[...] = _pack(_dot(hn, whs_ref[...]) + _dot(ni, wxb_ref[...]))


def _dec_body(hr_ref, st_ref, ni_ref, w1m_ref, b1m_ref, w2m_ref, db2_ref,
              dt_ref, out_ref):
    hn = _norm(hr_ref, st_ref)
    c1f = _silu(_dot(hn, w1m_ref[...]) + b1m_ref[...])
    diff = _dot(c1f, w2m_ref[...]) + db2_ref[...]
    steps = (lax.broadcasted_iota(jnp.int32, (1, TW), 1) + 1
             ).astype(jnp.float32)
    dtv = dt_ref[...] * steps
    out_ref[...] = ni_ref[:, TW - 1:TW] + dtv * diff


def _pre_call(x, pos, time, variables, *weights):
    pmax = pl.pallas_call(
        _max_body,
        out_shape=jax.ShapeDtypeStruct((1, 1), jnp.float32),
    )(pos)
    nrow = lambda w: pl.BlockSpec((BN, w), lambda j: (j, 0))
    full = lambda a, b: pl.BlockSpec((a, b), lambda j: (0, 0))
    return pl.pallas_call(
        _pre_body,
        grid=(NB,),
        in_specs=[
            nrow(TW), nrow(1), nrow(1), nrow(NV - 1), full(1, 1),
            full(TW + NS + NV, EMB), full(1, EMB), full(EMB, EMB),
            full(1, EMB), full(EMB, DP), full(TW + NS + NV, DP), full(1, DP),
            full(EMB, DP), full(TW + NS + NV, DP),
        ],
        out_specs=(nrow(TW + NS + NV), nrow(EMB), nrow(SP), nrow(SP)),
        out_shape=(jax.ShapeDtypeStruct((N, TW + NS + NV), jnp.float32),
                   jax.ShapeDtypeStruct((N, EMB), jnp.float32),
                   jax.ShapeDtypeStruct((N, SP), jnp.uint32),
                   jax.ShapeDtypeStruct((N, SP), jnp.uint32)),
    )(x, pos, time, variables, pmax, *weights)


def _msg_call(g, w2a, w2b, b2):
    nblk = g.shape[0] // BE
    return pl.pallas_call(
        _msg_body,
        grid=(nblk,),
        in_specs=[
            pl.BlockSpec((BE, SP), lambda i: (i, 0)),
            pl.BlockSpec((SP, DP), lambda i: (0, 0)),
            pl.BlockSpec((SP, DP), lambda i: (0, 0)),
            pl.BlockSpec((1, DP), lambda i: (0, 0)),
        ],
        out_specs=(pl.BlockSpec((BE, SP), lambda i: (i, 0)),
                   pl.BlockSpec((BE, SP), lambda i: (i, 0))),
        out_shape=(jax.ShapeDtypeStruct((g.shape[0], SP), jnp.float32),
                   jax.ShapeDtypeStruct((g.shape[0], SP), jnp.float32)),
    )(g, w2a, w2b, b2)


def _upd_call(h, los, his, ni, u1h, u1a, wxu, ub1, u2, ub2):
    nrow = lambda w: pl.BlockSpec((BN, w), lambda j: (j, 0))
    full = lambda a, b: pl.BlockSpec((a, b), lambda j: (0, 0))
    nagg = 2 * NSEG
    return pl.pallas_call(
        _upd_body,
        grid=(NB,),
        in_specs=[nrow(EMB)] + [nrow(SP)] * (2 * nagg) + [
            nrow(TW + NS + NV),
            full(EMB, EMB), full(EMB, EMB), full(TW + NS + NV, EMB),
            full(1, EMB), full(EMB, EMB), full(1, EMB),
        ],
        out_specs=(nrow(EMB), full(2, EMB)),
        out_shape=(jax.ShapeDtypeStruct((N, EMB), jnp.float32),
                   jax.ShapeDtypeStruct((2, EMB), jnp.float32)),
    )(h, *los, *his, ni, u1h, u1a, wxu, ub1, u2, ub2)


def _ab_call(hr, st, ni, whd, wxa, bm1, whs, wxb):
    nrow = lambda w: pl.BlockSpec((BN, w), lambda j: (j, 0))
    full = lambda a, b: pl.BlockSpec((a, b), lambda j: (0, 0))
    return pl.pallas_call(
        _ab_body,
        grid=(NB,),
        in_specs=[
            nrow(EMB), full(2, EMB), nrow(TW + NS + NV),
            full(EMB, DP), full(TW + NS + NV, DP), full(1, DP),
            full(EMB, DP), full(TW + NS + NV, DP),
        ],
        out_specs=(nrow(EMB), nrow(SP), nrow(SP)),
        out_shape=(jax.ShapeDtypeStruct((N, EMB), jnp.float32),
                   jax.ShapeDtypeStruct((N, SP), jnp.uint32),
                   jax.ShapeDtypeStruct((N, SP), jnp.uint32)),
    )(hr, st, ni, whd, wxa, bm1, whs, wxb)


def _dec_call(hr, st, ni, w1m, b1m, w2m, db2, dt2):
    nrow = lambda w: pl.BlockSpec((BN, w), lambda j: (j, 0))
    full = lambda a, b: pl.BlockSpec((a, b), lambda j: (0, 0))
    return pl.pallas_call(
        _dec_body,
        grid=(NB,),
        in_specs=[
            nrow(EMB), full(2, EMB), nrow(TW + NS + NV),
            full(EMB, 400), full(1, 400), full(400, TW),
            full(1, 1), full(1, 1),
        ],
        out_specs=nrow(TW),
        out_shape=jax.ShapeDtypeStruct((N, TW), jnp.float32),
    )(hr, st, ni, w1m, b1m, w2m, db2, dt2)


def kernel(x, pos, time, variables, batch, edge_index, dt, enc_W1, enc_b1,
           enc_W2, enc_b2, msg1_W, msg1_b, msg2_W, msg2_b, upd1_W, upd1_b,
           upd2_W, upd2_b, dec1_W, dec1_b, dec2_W, dec2_b):
    f32 = jnp.float32
    padc = lambda w: jnp.pad(w, [(0, 0)] * (w.ndim - 1) + [(0, DP - EMB)])
    Whd = padc(msg1_W[:, :EMB, :])
    Whs = padc(msg1_W[:, EMB:2 * EMB, :])
    WxA = padc(msg1_W[:, 2 * EMB:, :])
    WxB = padc(jnp.concatenate(
        [-msg1_W[:, 2 * EMB:2 * EMB + TW + NS, :],
         jnp.zeros((L, NV, EMB), f32)], axis=1))
    bm1 = padc(msg1_b)[:, None, :]
    W2p = jnp.pad(msg2_W, ((0, 0), (0, DP - EMB), (0, DP - EMB)))
    b2p = padc(msg2_b)[:, None, :]
    U1h = upd1_W[:, :EMB, :]
    U1a = upd1_W[:, EMB:2 * EMB, :]
    WxU = jnp.concatenate(
        [jnp.zeros((L, TW + NS, EMB), f32), upd1_W[:, 2 * EMB:, :]], axis=1)
    ub1 = upd1_b[:, None, :]
    ub2 = upd2_b[:, None, :]
    o_id = jnp.repeat(jnp.arange(8), 50)
    j_id = jnp.tile(jnp.arange(50), 8)
    fgrid = jnp.arange(EMB)[:, None]
    k1 = fgrid - 3 * j_id[None, :]
    W1m = jnp.where((k1 >= 0) & (k1 < 16),
                    dec1_W[o_id[None, :], 0, jnp.clip(k1, 0, 15)], 0.0)
    b1m = dec1_b[o_id][None, :]
    k2 = j_id[:, None] - 9 * jnp.arange(5)[None, :]
    W2m = jnp.where((k2 >= 0) & (k2 < 14),
                    dec2_W[0, o_id[:, None], jnp.clip(k2, 0, 13)], 0.0)
    db2 = dec2_b[None, :]
    dt2 = dt[None, :]

    src = edge_index[0]
    dst = edge_index[1]
    EPH = EPAD // NSEG
    src_g = jnp.pad(src, (0, EPAD - E)).reshape(EPAD // CHG, CHG)
    dst_g = jnp.pad(dst, (0, EPAD - E)).reshape(EPAD // CHG, CHG)
    dst_s = jnp.pad(dst, (0, EPAD - E),
                    constant_values=N).reshape(EPAD // CH, CH)
    hg = EPH // CHG
    hs = EPH // CH
    r3 = lambda a, n: a.reshape(NW, n // NW, a.shape[-1])
    src_gs = [r3(src_g[q * hg:(q + 1) * hg], hg) for q in range(NSEG)]
    dst_gs = [r3(dst_g[q * hg:(q + 1) * hg], hg) for q in range(NSEG)]
    dst_ss = [r3(dst_s[q * hs:(q + 1) * hs], hs) for q in range(NSEG)]
    zeros = jnp.zeros((NP, SP), f32)

    ni, h, A, B = _pre_call(x, pos, time, variables, enc_W1, enc_b1[None, :],
                            enc_W2, enc_b2[None, :], Whd[0], WxA[0], bm1[0],
                            Whs[0], WxB[0])
    sc_gather, sc_scatter = _sc_kernels(EPH)
    out = None
    for i in range(L):
        gs = [sc_gather(A, B, dst_gs[q], src_gs[q]) for q in range(NSEG)]
        los, his = [], []
        for q in range(NSEG):
            s1q, s2q = _msg_call(gs[q], W2p[i, :SP], W2p[i, SP:], b2p[i])
            agq = sc_scatter(s1q, s2q, dst_ss[q], zeros)
            los += [agq[0, 0], agq[1, 0]]
            his += [agq[0, 1], agq[1, 1]]
        hr, st = _upd_call(h, los, his,
                           ni, U1h[i], U1a[i], WxU[i], ub1[i], upd2_W[i],
                           ub2[i])
        if i < L - 1:
            h, A, B = _ab_call(hr, st, ni, Whd[i + 1], WxA[i + 1], bm1[i + 1],
                               Whs[i + 1], WxB[i + 1])
        else:
            out = _dec_call(hr, st, ni, W1m, b1m, W2m, db2, dt2)
    return out


# speedup vs baseline: 1.0117x; 1.0002x over previous
import functools

import jax
import jax.numpy as jnp
from jax import lax
from jax.experimental import pallas as pl
from jax.experimental.pallas import tpu as pltpu
from jax.experimental.pallas import tpu_sc as plsc

N = 10000
E = 160000
TW = 5
NV = 2
NS = 1
EMB = 164
L = 6

DP = 256
SP = 128
NP = N + 8
NC = 2
NSUB = 16
NW = NC * NSUB
CH = 128
CHG = 64
EPAD = 163840
EPW = EPAD // NW
NCHUNK = EPW // CH
NCHUNKG = EPW // CHG
BE = 2048
BN = 2000
NB = N // BN


def _silu(v):
    return v * jax.nn.sigmoid(v)


def _dot(a, b):
    return jnp.dot(a, b, preferred_element_type=jnp.float32)


@functools.lru_cache(maxsize=None)
def _sc_kernels(epad):
    mesh = plsc.VectorSubcoreMesh(core_axis_name="c", subcore_axis_name="s")
    epw = epad // NW
    nchg = epw // CHG
    nch = epw // CH

    @functools.partial(
        pl.kernel,
        mesh=mesh,
        compiler_params=pltpu.CompilerParams(needs_layout_passes=False),
        out_type=jax.ShapeDtypeStruct((epad, SP), jnp.uint32),
        scratch_types=[
            pltpu.VMEM((nchg, CHG), jnp.int32),
            pltpu.VMEM((nchg, CHG), jnp.int32),
            pltpu.VMEM((4, CHG, SP), jnp.uint32),
            pltpu.VMEM((4, CHG, SP), jnp.uint32),
            pltpu.SemaphoreType.DMA,
            pltpu.SemaphoreType.DMA,
            pltpu.SemaphoreType.DMA,
            pltpu.SemaphoreType.DMA,
            pltpu.SemaphoreType.DMA,
            pltpu.SemaphoreType.DMA,
            pltpu.SemaphoreType.DMA,
            pltpu.SemaphoreType.DMA,
        ],
    )
    def sc_gather(a_hbm, b_hbm, dst2_hbm, src2_hbm, g_hbm,
                  dsti, srci, bufa, bufb,
                  sa0, sb0, sa1, sb1, sa2, sb2, sa3, sb3):
        c = lax.axis_index("c")
        s = lax.axis_index("s")
        wid = s * NC + c
        base = wid * epw
        sems_a = (sa0, sa1, sa2, sa3)
        sems_b = (sb0, sb1, sb2, sb3)
        pltpu.sync_copy(dst2_hbm.at[wid], dsti)
        pltpu.sync_copy(src2_hbm.at[wid], srci)

        def fire(k, p):
            pltpu.async_copy(a_hbm.at[dsti.at[k]], bufa.at[p], sems_a[p])
            pltpu.async_copy(b_hbm.at[srci.at[k]], bufb.at[p], sems_b[p])

        def process(k, p):
            pltpu.make_async_copy(
                a_hbm.at[dsti.at[k]], bufa.at[p], sems_a[p]).wait()
            pltpu.make_async_copy(
                b_hbm.at[srci.at[k]], bufb.at[p], sems_b[p]).wait()

            def add_row(r2, carry):
                for rr in range(2):
                    r = 2 * r2 + rr
                    for j in range(SP // 16):
                        sl = pl.ds(j * 16, 16)
                        va = plsc.bitcast(bufa[p, r, sl], jnp.bfloat16)
                        vb = plsc.bitcast(bufb[p, r, sl], jnp.bfloat16)
                        bufa[p, r, sl] = plsc.bitcast(va + vb, jnp.uint32)
                return carry

            lax.fori_loop(0, CHG // 2, add_row, 0)
            pltpu.sync_copy(bufa.at[p],
                            g_hbm.at[pl.ds(base + k * CHG, CHG)])

        fire(0, 0)
        fire(1, 1)
        fire(2, 2)
        last = nchg // 4 - 1

        def body(k4, carry):
            e = 4 * k4
            fire(e + 3, 3)
            process(e, 0)

            @pl.when(k4 < last)
            def _f0():
                fire(e + 4, 0)

            process(e + 1, 1)

            @pl.when(k4 < last)
            def _f1():
                fire(e + 5, 1)

            process(e + 2, 2)

            @pl.when(k4 < last)
            def _f2():
                fire(e + 6, 2)

            process(e + 3, 3)
            return carry

        lax.fori_loop(0, nchg // 4, body, 0)

    @functools.partial(
        pl.kernel,
        mesh=mesh,
        out_type=jax.ShapeDtypeStruct((NC, 2, NP, SP), jnp.float32),
        scratch_types=[
            pltpu.VMEM((nch, CH), jnp.int32),
            pltpu.VMEM((2, CH, SP), jnp.float32),
            pltpu.VMEM_SHARED((NP, SP), jnp.float32),
            pltpu.SemaphoreType.DMA,
            pltpu.SemaphoreType.DMA,
        ],
    )
    def sc_scatter(s1_hbm, s2_hbm, dst2s_hbm, zeros_hbm, out_hbm,
                   idxv, bufs, shared, se0, se1):
        c = lax.axis_index("c")
        s = lax.axis_index("s")
        wid = s * NC + c
        base = wid * epw
        sems = (se0, se1)
        pltpu.sync_copy(dst2s_hbm.at[wid], idxv)
        for src_hbm, pp in ((s1_hbm, 0), (s2_hbm, 1)):
            @pl.when(s == 0)
            def _zero():
                pltpu.sync_copy(zeros_hbm, shared)

            plsc.subcore_barrier()

            def fire(k, q, src_hbm=src_hbm):
                pltpu.async_copy(src_hbm.at[pl.ds(base + k * CH, CH)],
                                 bufs.at[q], sems[q])

            def process(k, q, src_hbm=src_hbm):
                pltpu.make_async_copy(
                    src_hbm.at[pl.ds(base + k * CH, CH)],
                    bufs.at[q], sems[q]).wait()
                pltpu.sync_copy(bufs.at[q], shared.at[idxv.at[k]], add=True)

            fire(0, 0)

            def body(k2, carry):
                e = 2 * k2
                fire(e + 1, 1)
                process(e, 0)

                @pl.when(k2 < nch // 2 - 1)
                def _refire():
                    fire(e + 2, 0)

                process(e + 1, 1)
                return carry

            lax.fori_loop(0, nch // 2, body, 0)
            plsc.subcore_barrier()

            @pl.when(s == 0)
            def _dump(pp=pp):
                pltpu.sync_copy(shared, out_hbm.at[c, pp])

    return sc_gather, sc_scatter


def _max_body(pos_ref, out_ref):
    out_ref[...] = jnp.max(pos_ref[...]).reshape(1, 1)


def _pack(v):
    vb = v.astype(jnp.bfloat16)
    lo = jax.lax.bitcast_convert_type(vb[:, :SP], jnp.uint16
                                      ).astype(jnp.uint32)
    hi = jax.lax.bitcast_convert_type(vb[:, SP:], jnp.uint16
                                      ).astype(jnp.uint32)
    return lo | (hi << jnp.uint32(16))


def _pre_body(x_ref, pos_ref, time_ref, vars_ref, pmax_ref, w1_ref, b1_ref,
              w2_ref, b2_ref, whd_ref, wxa_ref, bm1_ref, whs_ref, wxb_ref,
              ni_ref, h_ref, a_ref, b_ref):
    pos_n = pos_ref[...] / pmax_ref[...]
    ni = jnp.concatenate(
        [x_ref[...], pos_n, time_ref[...] * (1.0 / 10.0), vars_ref[...]],
        axis=1)
    h = _silu(_dot(ni, w1_ref[...]) + b1_ref[...])
    h = _silu(_dot(h, w2_ref[...]) + b2_ref[...])
    ni_ref[...] = ni
    h_ref[...] = h
    a_ref[...] = _pack(_dot(h, whd_ref[...]) + _dot(ni, wxa_ref[...])
                       + bm1_ref[...])
    b_ref[...] = _pack(_dot(h, whs_ref[...]) + _dot(ni, wxb_ref[...]))


def _unpk(bits16):
    return jax.lax.bitcast_convert_type(
        bits16.astype(jnp.uint16), jnp.bfloat16).astype(jnp.float32)


def _msg_body(g_ref, w2a_ref, w2b_ref, b2_ref, s1_ref, s2_ref):
    u = g_ref[...]
    tl = _silu(_unpk(u & jnp.uint32(0xFFFF)))
    th = _silu(_unpk(u >> jnp.uint32(16)))
    sv = _silu(_dot(tl, w2a_ref[...]) + _dot(th, w2b_ref[...])
               + b2_ref[...])
    col = lax.broadcasted_iota(jnp.int32, sv.shape, 1)
    sv = jnp.where(col == EMB, 1.0, sv)
    s1_ref[...] = sv[:, :SP]
    s2_ref[...] = sv[:, SP:2 * SP]


NSEG = 2


def _upd_body(*refs):
    h_ref = refs[0]
    nagg = 2 * NSEG
    los = refs[1:1 + nagg]
    his = refs[1 + nagg:1 + 2 * nagg]
    (ni_ref, u1h_ref, u1a_ref, wxu_ref, ub1_ref, u2_ref, ub2_ref,
     hr_ref, st_ref) = refs[1 + 2 * nagg:]
    j = pl.program_id(0)
    h = h_ref[...]
    dcol = EMB - SP
    lo = los[0][...]
    for r in los[1:]:
        lo = lo + r[...]
    hi = his[0][...]
    for r in his[1:]:
        hi = hi + r[...]
    deg = jnp.clip(hi[:, dcol:dcol + 1], 1.0, None)
    agg = jnp.concatenate([lo, hi[:, :dcol]], axis=1) / deg
    u = _silu(_dot(h, u1h_ref[...]) + _dot(agg, u1a_ref[...]) +
              _dot(ni_ref[...], wxu_ref[...]) + ub1_ref[...])
    u = _silu(_dot(u, u2_ref[...]) + ub2_ref[...])
    hr = h + u
    hr_ref[...] = hr

    @pl.when(j == 0)
    def _init():
        st_ref[...] = jnp.zeros_like(st_ref)

    st_ref[...] += jnp.concatenate(
        [jnp.sum(hr, axis=0, keepdims=True),
         jnp.sum(hr * hr, axis=0, keepdims=True)], axis=0)


def _norm(hr_ref, st_ref):
    mean = st_ref[0:1, :] * (1.0 / N)
    ex2 = st_ref[1:2, :] * (1.0 / N)
    varr = jnp.maximum(ex2 - mean * mean, 0.0)
    return (hr_ref[...] - mean) * lax.rsqrt(varr + 1e-5)


def _ab_body(hr_ref, st_ref, ni_ref, whd_ref, wxa_ref, bm1_ref, whs_ref,
             wxb_ref, hn_ref, an_ref, bn_ref):
    hn = _norm(hr_ref, st_ref)
    ni = ni_ref[...]
    hn_ref[...] = hn
    an_ref[...] = _pack(_dot(hn, whd_ref[...]) + _dot(ni, wxa_ref[...])
                        + bm1_ref[...])
    bn_ref[...] = _pack(_dot(hn, whs_ref[...]) + _dot(ni, wxb_ref[...]))


def _dec_body(hr_ref, st_ref, ni_ref, w1m_ref, b1m_ref, w2m_ref, db2_ref,
              dt_ref, out_ref):
    hn = _norm(hr_ref, st_ref)
    c1f = _silu(_dot(hn, w1m_ref[...]) + b1m_ref[...])
    diff = _dot(c1f, w2m_ref[...]) + db2_ref[...]
    steps = (lax.broadcasted_iota(jnp.int32, (1, TW), 1) + 1
             ).astype(jnp.float32)
    dtv = dt_ref[...] * steps
    out_ref[...] = ni_ref[:, TW - 1:TW] + dtv * diff


def _pre_call(x, pos, time, variables, *weights):
    pmax = pl.pallas_call(
        _max_body,
        out_shape=jax.ShapeDtypeStruct((1, 1), jnp.float32),
    )(pos)
    nrow = lambda w: pl.BlockSpec((BN, w), lambda j: (j, 0))
    full = lambda a, b: pl.BlockSpec((a, b), lambda j: (0, 0))
    return pl.pallas_call(
        _pre_body,
        grid=(NB,),
        in_specs=[
            nrow(TW), nrow(1), nrow(1), nrow(NV - 1), full(1, 1),
            full(TW + NS + NV, EMB), full(1, EMB), full(EMB, EMB),
            full(1, EMB), full(EMB, DP), full(TW + NS + NV, DP), full(1, DP),
            full(EMB, DP), full(TW + NS + NV, DP),
        ],
        out_specs=(nrow(TW + NS + NV), nrow(EMB), nrow(SP), nrow(SP)),
        out_shape=(jax.ShapeDtypeStruct((N, TW + NS + NV), jnp.float32),
                   jax.ShapeDtypeStruct((N, EMB), jnp.float32),
                   jax.ShapeDtypeStruct((N, SP), jnp.uint32),
                   jax.ShapeDtypeStruct((N, SP), jnp.uint32)),
    )(x, pos, time, variables, pmax, *weights)


def _msg_call(g, w2a, w2b, b2):
    nblk = g.shape[0] // BE
    return pl.pallas_call(
        _msg_body,
        grid=(nblk,),
        in_specs=[
            pl.BlockSpec((BE, SP), lambda i: (i, 0)),
            pl.BlockSpec((SP, DP), lambda i: (0, 0)),
            pl.BlockSpec((SP, DP), lambda i: (0, 0)),
            pl.BlockSpec((1, DP), lambda i: (0, 0)),
        ],
        out_specs=(pl.BlockSpec((BE, SP), lambda i: (i, 0)),
                   pl.BlockSpec((BE, SP), lambda i: (i, 0))),
        out_shape=(jax.ShapeDtypeStruct((g.shape[0], SP), jnp.float32),
                   jax.ShapeDtypeStruct((g.shape[0], SP), jnp.float32)),
    )(g, w2a, w2b, b2)


def _upd_call(h, los, his, ni, u1h, u1a, wxu, ub1, u2, ub2):
    nrow = lambda w: pl.BlockSpec((BN, w), lambda j: (j, 0))
    full = lambda a, b: pl.BlockSpec((a, b), lambda j: (0, 0))
    nagg = 2 * NSEG
    return pl.pallas_call(
        _upd_body,
        grid=(NB,),
        in_specs=[nrow(EMB)] + [nrow(SP)] * (2 * nagg) + [
            nrow(TW + NS + NV),
            full(EMB, EMB), full(EMB, EMB), full(TW + NS + NV, EMB),
            full(1, EMB), full(EMB, EMB), full(1, EMB),
        ],
        out_specs=(nrow(EMB), full(2, EMB)),
        out_shape=(jax.ShapeDtypeStruct((N, EMB), jnp.float32),
                   jax.ShapeDtypeStruct((2, EMB), jnp.float32)),
    )(h, *los, *his, ni, u1h, u1a, wxu, ub1, u2, ub2)


def _ab_call(hr, st, ni, whd, wxa, bm1, whs, wxb):
    nrow = lambda w: pl.BlockSpec((BN, w), lambda j: (j, 0))
    full = lambda a, b: pl.BlockSpec((a, b), lambda j: (0, 0))
    return pl.pallas_call(
        _ab_body,
        grid=(NB,),
        in_specs=[
            nrow(EMB), full(2, EMB), nrow(TW + NS + NV),
            full(EMB, DP), full(TW + NS + NV, DP), full(1, DP),
            full(EMB, DP), full(TW + NS + NV, DP),
        ],
        out_specs=(nrow(EMB), nrow(SP), nrow(SP)),
        out_shape=(jax.ShapeDtypeStruct((N, EMB), jnp.float32),
                   jax.ShapeDtypeStruct((N, SP), jnp.uint32),
                   jax.ShapeDtypeStruct((N, SP), jnp.uint32)),
    )(hr, st, ni, whd, wxa, bm1, whs, wxb)


def _dec_call(hr, st, ni, w1m, b1m, w2m, db2, dt2):
    nrow = lambda w: pl.BlockSpec((BN, w), lambda j: (j, 0))
    full = lambda a, b: pl.BlockSpec((a, b), lambda j: (0, 0))
    return pl.pallas_call(
        _dec_body,
        grid=(NB,),
        in_specs=[
            nrow(EMB), full(2, EMB), nrow(TW + NS + NV),
            full(EMB, 400), full(1, 400), full(400, TW),
            full(1, 1), full(1, 1),
        ],
        out_specs=nrow(TW),
        out_shape=jax.ShapeDtypeStruct((N, TW), jnp.float32),
    )(hr, st, ni, w1m, b1m, w2m, db2, dt2)


def kernel(x, pos, time, variables, batch, edge_index, dt, enc_W1, enc_b1,
           enc_W2, enc_b2, msg1_W, msg1_b, msg2_W, msg2_b, upd1_W, upd1_b,
           upd2_W, upd2_b, dec1_W, dec1_b, dec2_W, dec2_b):
    f32 = jnp.float32
    padc = lambda w: jnp.pad(w, [(0, 0)] * (w.ndim - 1) + [(0, DP - EMB)])
    Whd = padc(msg1_W[:, :EMB, :])
    Whs = padc(msg1_W[:, EMB:2 * EMB, :])
    WxA = padc(msg1_W[:, 2 * EMB:, :])
    WxB = padc(jnp.concatenate(
        [-msg1_W[:, 2 * EMB:2 * EMB + TW + NS, :],
         jnp.zeros((L, NV, EMB), f32)], axis=1))
    bm1 = padc(msg1_b)[:, None, :]
    W2p = jnp.pad(msg2_W, ((0, 0), (0, DP - EMB), (0, DP - EMB)))
    b2p = padc(msg2_b)[:, None, :]
    U1h = upd1_W[:, :EMB, :]
    U1a = upd1_W[:, EMB:2 * EMB, :]
    WxU = jnp.concatenate(
        [jnp.zeros((L, TW + NS, EMB), f32), upd1_W[:, 2 * EMB:, :]], axis=1)
    ub1 = upd1_b[:, None, :]
    ub2 = upd2_b[:, None, :]
    o_id = jnp.repeat(jnp.arange(8), 50)
    j_id = jnp.tile(jnp.arange(50), 8)
    fgrid = jnp.arange(EMB)[:, None]
    k1 = fgrid - 3 * j_id[None, :]
    W1m = jnp.where((k1 >= 0) & (k1 < 16),
                    dec1_W[o_id[None, :], 0, jnp.clip(k1, 0, 15)], 0.0)
    b1m = dec1_b[o_id][None, :]
    k2 = j_id[:, None] - 9 * jnp.arange(5)[None, :]
    W2m = jnp.where((k2 >= 0) & (k2 < 14),
                    dec2_W[0, o_id[:, None], jnp.clip(k2, 0, 13)], 0.0)
    db2 = dec2_b[None, :]
    dt2 = dt[None, :]

    src = edge_index[0]
    dst = edge_index[1]
    EPH = EPAD // NSEG
    src_g = jnp.pad(src, (0, EPAD - E)).reshape(EPAD // CHG, CHG)
    dst_g = jnp.pad(dst, (0, EPAD - E)).reshape(EPAD // CHG, CHG)
    dst_s = jnp.pad(dst, (0, EPAD - E),
                    constant_values=N).reshape(EPAD // CH, CH)
    hg = EPH // CHG
    hs = EPH // CH
    r3 = lambda a, n: a.reshape(NW, n // NW, a.shape[-1])
    src_gs = [r3(src_g[q * hg:(q + 1) * hg], hg) for q in range(NSEG)]
    dst_gs = [r3(dst_g[q * hg:(q + 1) * hg], hg) for q in range(NSEG)]
    dst_ss = [r3(dst_s[q * hs:(q + 1) * hs], hs) for q in range(NSEG)]
    zeros = jnp.zeros((NP, SP), f32)

    ni, h, A, B = _pre_call(x, pos, time, variables, enc_W1, enc_b1[None, :],
                            enc_W2, enc_b2[None, :], Whd[0], WxA[0], bm1[0],
                            Whs[0], WxB[0])
    sc_gather, sc_scatter = _sc_kernels(EPH)
    out = None
    for i in range(L):
        gs = [sc_gather(A, B, dst_gs[q], src_gs[q]) for q in range(NSEG)]
        los, his = [], []
        for q in range(NSEG):
            s1q, s2q = _msg_call(gs[q], W2p[i, :SP], W2p[i, SP:], b2p[i])
            agq = sc_scatter(s1q, s2q, dst_ss[q], zeros)
            los += [agq[0, 0], agq[1, 0]]
            his += [agq[0, 1], agq[1, 1]]
        hr, st = _upd_call(h, los, his,
                           ni, U1h[i], U1a[i], WxU[i], ub1[i], upd2_W[i],
                           ub2[i])
        if i < L - 1:
            h, A, B = _ab_call(hr, st, ni, Whd[i + 1], WxA[i + 1], bm1[i + 1],
                               Whs[i + 1], WxB[i + 1])
        else:
            out = _dec_call(hr, st, ni, W1m, b1m, W2m, db2, dt2)
    return out
